# baseline (speedup 1.0000x reference)
import numpy as np

import jax
import jax.numpy as jnp
from jax import lax
from jax.experimental import pallas as pl
from jax.experimental.pallas import tpu as pltpu
from jax.experimental.pallas import tpu_sc as plsc

_ALPHA = 0.2
_LAM = float(np.random.RandomState(0).beta(_ALPHA, _ALPHA))

_NSUB = 16
_GPR = 128 // _NSUB
_ROWS_PER_W = 2

with jax.default_device(jax.devices("cpu")[0]):
    _PERM_NP = np.asarray(
        jax.random.permutation(jax.random.key(42), 64)).astype(np.int32)


def _sc_mix_body(x_hbm, out_hbm, a0, a1, b0, b1, o0, o1, sa, sb, so):
    nc = 2
    wid = lax.axis_index("s") * nc + lax.axis_index("c")

    abufs = (a0, a1)
    bbufs = (b0, b1)
    obufs = (o0, o1)

    rows = []
    for k in range(_ROWS_PER_W):
        r = wid * _ROWS_PER_W + k
        q = jnp.int32(_PERM_NP[k])
        for w in range(32):
            q = jnp.where(wid == w, jnp.int32(_PERM_NP[w * _ROWS_PER_W + k]), q)
        rows.append((r, q))

    steps = [(k, g) for k in range(_ROWS_PER_W) for g in range(_GPR)]
    n = len(steps)

    def issue_in(gg):
        k, g = steps[gg]
        r, q = rows[k]
        ha = pltpu.async_copy(
            x_hbm.at[r, pl.ds(g * _NSUB, _NSUB)], abufs[gg % 2], sa.at[gg % 2])
        hb = pltpu.async_copy(
            x_hbm.at[q, pl.ds(g * _NSUB, _NSUB)], bbufs[gg % 2], sb.at[gg % 2])
        return (ha, hb)

    def issue_out(gg):
        k, g = steps[gg]
        r, _ = rows[k]
        return pltpu.async_copy(
            obufs[gg % 2], out_hbm.at[r, pl.ds(g * _NSUB, _NSUB)], so.at[gg % 2])

    in_h = [None] * n
    out_h = [None] * n
    in_h[0] = issue_in(0)

    for gg in range(n):
        if gg + 1 < n:
            in_h[gg + 1] = issue_in(gg + 1)
        in_h[gg][0].wait()
        in_h[gg][1].wait()
        if gg >= 2:
            out_h[gg - 2].wait()
        a, b, o = abufs[gg % 2], bbufs[gg % 2], obufs[gg % 2]

        @plsc.parallel_loop(0, _NSUB * 64, unroll=16)
        def _blend(i):
            s = i // 64
            col = (i % 64) * 16
            sl = pl.ds(col, 16)
            o[s, sl] = _LAM * a[s, sl] + (1.0 - _LAM) * b[s, sl]
        out_h[gg] = issue_out(gg)

    out_h[n - 2].wait()
    out_h[n - 1].wait()


def _lab_kernel(perm_ref, l_ref, ol_ref):
    i = pl.program_id(0)
    j = perm_ref[i]
    ol_ref[0, 0] = _LAM * l_ref[i, 0] + (1.0 - _LAM) * l_ref[j, 0]


def kernel(batch_spectrograms, batch_labels):
    B, C, H, W = batch_spectrograms.shape
    L = batch_labels.shape[1]
    perm = jax.random.permutation(jax.random.key(42), B).astype(jnp.int32)

    x3 = batch_spectrograms.reshape(B, H, W)

    mesh = plsc.VectorSubcoreMesh(core_axis_name="c", subcore_axis_name="s")
    sc_call = pl.kernel(
        _sc_mix_body,
        mesh=mesh,
        out_type=jax.ShapeDtypeStruct((B, H, W), jnp.float32),
        scratch_types=[
            pltpu.VMEM((_NSUB, W), jnp.float32),
            pltpu.VMEM((_NSUB, W), jnp.float32),
            pltpu.VMEM((_NSUB, W), jnp.float32),
            pltpu.VMEM((_NSUB, W), jnp.float32),
            pltpu.VMEM((_NSUB, W), jnp.float32),
            pltpu.VMEM((_NSUB, W), jnp.float32),
            pltpu.SemaphoreType.DMA((2,)),
            pltpu.SemaphoreType.DMA((2,)),
            pltpu.SemaphoreType.DMA((2,)),
        ],
    )
    ox = sc_call(x3).reshape(B, C, H, W)

    labels3 = batch_labels[:, None, :]
    grid_spec = pltpu.PrefetchScalarGridSpec(
        num_scalar_prefetch=1,
        grid=(B,),
        in_specs=[pl.BlockSpec((B, 1, L), lambda g, p: (0, 0, 0))],
        out_specs=[pl.BlockSpec((1, 1, L), lambda g, p: (g, 0, 0))],
    )
    ol = pl.pallas_call(
        _lab_kernel,
        grid_spec=grid_spec,
        out_shape=[jax.ShapeDtypeStruct(labels3.shape, jnp.float32)],
    )(perm, labels3)[0]
    return ox, ol[:, 0, :]


# speedup vs baseline: 1.1971x; 1.1971x over previous
import numpy as np

import jax
import jax.numpy as jnp
from jax import lax
from jax.experimental import pallas as pl
from jax.experimental.pallas import tpu as pltpu
from jax.experimental.pallas import tpu_sc as plsc

_ALPHA = 0.2
_LAM = float(np.random.RandomState(0).beta(_ALPHA, _ALPHA))

_NCHUNK = 16

with jax.default_device(jax.devices("cpu")[0]):
    _PERM_NP = np.asarray(
        jax.random.permutation(jax.random.key(42), 64)).astype(np.int32)

_ROWS_PER_W = 2
_LPAD = 528


def _spec_kernel(order_ref, po_ref, needed_ref, x_hbm, ox_ref, buf, sems,
                 waited):
    g = pl.program_id(0)
    nrows = x_hbm.shape[0]
    rpc = nrows // _NCHUNK

    @pl.when(g == 0)
    def _():
        waited[0] = 0
        for c in range(_NCHUNK):
            pltpu.make_async_copy(
                x_hbm.at[pl.ds(c * rpc, rpc)],
                buf.at[pl.ds(c * rpc, rpc)],
                sems.at[c],
            ).start()

    need = needed_ref[g]
    w0 = waited[0]
    for c in range(_NCHUNK):
        @pl.when(jnp.logical_and(c >= w0, c <= need))
        def _(c=c):
            pltpu.make_async_copy(
                x_hbm.at[pl.ds(c * rpc, rpc)],
                buf.at[pl.ds(c * rpc, rpc)],
                sems.at[c],
            ).wait()
    waited[0] = jnp.maximum(w0, need + 1)

    i = order_ref[g]
    j = po_ref[g]
    ox_ref[0, 0] = _LAM * buf[i, 0] + (1.0 - _LAM) * buf[j, 0]


def _spec_mix(batch_spectrograms):
    B, C, H, W = batch_spectrograms.shape
    rpc = B // _NCHUNK
    perm = jnp.asarray(_PERM_NP)

    rows_np = np.arange(B, dtype=np.int32)
    last_chunk = np.maximum(rows_np // rpc, _PERM_NP // rpc)
    order_np = np.argsort(last_chunk, kind="stable").astype(np.int32)
    order = jnp.asarray(order_np)
    po = jnp.asarray(_PERM_NP[order_np])
    needed = jnp.asarray(last_chunk[order_np].astype(np.int32))

    grid_spec = pltpu.PrefetchScalarGridSpec(
        num_scalar_prefetch=3,
        grid=(B,),
        in_specs=[pl.BlockSpec(memory_space=pl.ANY)],
        out_specs=[pl.BlockSpec((1, C, H, W), lambda g, o, p, n: (o[g], 0, 0, 0))],
        scratch_shapes=[
            pltpu.VMEM((B, C, H, W), jnp.float32),
            pltpu.SemaphoreType.DMA((_NCHUNK,)),
            pltpu.SMEM((1,), jnp.int32),
        ],
    )
    del perm
    return pl.pallas_call(
        _spec_kernel,
        grid_spec=grid_spec,
        out_shape=[jax.ShapeDtypeStruct(batch_spectrograms.shape, jnp.float32)],
    )(order, po, needed, batch_spectrograms)[0]


def _lab_sc_body(l_hbm, out_hbm, a, b, o, sa, sb, so):
    nc = 2
    wid = lax.axis_index("s") * nc + lax.axis_index("c")

    for k in range(_ROWS_PER_W):
        r = wid * _ROWS_PER_W + k
        q = jnp.int32(_PERM_NP[k])
        for w in range(32):
            q = jnp.where(wid == w, jnp.int32(_PERM_NP[w * _ROWS_PER_W + k]), q)

        ha = pltpu.async_copy(l_hbm.at[r], a, sa)
        hb = pltpu.async_copy(l_hbm.at[q], b, sb)
        ha.wait()
        hb.wait()

        @plsc.parallel_loop(0, _LPAD // 16, unroll=4)
        def _blend(i):
            sl = pl.ds(i * 16, 16)
            o[sl] = _LAM * a[sl] + (1.0 - _LAM) * b[sl]

        pltpu.async_copy(o, out_hbm.at[r], so).wait()


def _lab_mix(batch_labels):
    B, L = batch_labels.shape
    lp = jnp.pad(batch_labels, ((0, 0), (0, _LPAD - L)))
    mesh = plsc.VectorSubcoreMesh(core_axis_name="c", subcore_axis_name="s")
    out = pl.kernel(
        _lab_sc_body,
        mesh=mesh,
        out_type=jax.ShapeDtypeStruct((B, _LPAD), jnp.float32),
        scratch_types=[
            pltpu.VMEM((_LPAD,), jnp.float32),
            pltpu.VMEM((_LPAD,), jnp.float32),
            pltpu.VMEM((_LPAD,), jnp.float32),
            pltpu.SemaphoreType.DMA,
            pltpu.SemaphoreType.DMA,
            pltpu.SemaphoreType.DMA,
        ],
    )(lp)
    return out[:, :L]


def kernel(batch_spectrograms, batch_labels):
    ol = _lab_mix(batch_labels)
    ox = _spec_mix(batch_spectrograms)
    return ox, ol


# speedup vs baseline: 1.2031x; 1.0050x over previous
import numpy as np

import jax
import jax.numpy as jnp
from jax import lax
from jax.experimental import pallas as pl
from jax.experimental.pallas import tpu as pltpu
from jax.experimental.pallas import tpu_sc as plsc

_ALPHA = 0.2
_LAM = float(np.random.RandomState(0).beta(_ALPHA, _ALPHA))

_NCHUNK = 32

with jax.default_device(jax.devices("cpu")[0]):
    _PERM_NP = np.asarray(
        jax.random.permutation(jax.random.key(42), 64)).astype(np.int32)

_ROWS_PER_W = 2
_LPAD = 528


def _spec_kernel(order_ref, po_ref, needed_ref, x_hbm, ox_ref, buf, sems,
                 waited):
    g = pl.program_id(0)
    nrows = x_hbm.shape[0]
    rpc = nrows // _NCHUNK

    @pl.when(g == 0)
    def _():
        waited[0] = 0
        for c in range(_NCHUNK):
            pltpu.make_async_copy(
                x_hbm.at[pl.ds(c * rpc, rpc)],
                buf.at[pl.ds(c * rpc, rpc)],
                sems.at[c],
            ).start()

    need = needed_ref[g]
    w0 = waited[0]
    for c in range(_NCHUNK):
        @pl.when(jnp.logical_and(c >= w0, c <= need))
        def _(c=c):
            pltpu.make_async_copy(
                x_hbm.at[pl.ds(c * rpc, rpc)],
                buf.at[pl.ds(c * rpc, rpc)],
                sems.at[c],
            ).wait()
    waited[0] = jnp.maximum(w0, need + 1)

    i = order_ref[g]
    j = po_ref[g]
    ox_ref[0, 0] = _LAM * buf[i, 0] + (1.0 - _LAM) * buf[j, 0]


def _spec_mix(batch_spectrograms):
    B, C, H, W = batch_spectrograms.shape
    rpc = B // _NCHUNK
    perm = jnp.asarray(_PERM_NP)

    rows_np = np.arange(B, dtype=np.int32)
    last_chunk = np.maximum(rows_np // rpc, _PERM_NP // rpc)
    order_np = np.argsort(last_chunk, kind="stable").astype(np.int32)
    order = jnp.asarray(order_np)
    po = jnp.asarray(_PERM_NP[order_np])
    needed = jnp.asarray(last_chunk[order_np].astype(np.int32))

    grid_spec = pltpu.PrefetchScalarGridSpec(
        num_scalar_prefetch=3,
        grid=(B,),
        in_specs=[pl.BlockSpec(memory_space=pl.ANY)],
        out_specs=[pl.BlockSpec((1, C, H, W), lambda g, o, p, n: (o[g], 0, 0, 0))],
        scratch_shapes=[
            pltpu.VMEM((B, C, H, W), jnp.float32),
            pltpu.SemaphoreType.DMA((_NCHUNK,)),
            pltpu.SMEM((1,), jnp.int32),
        ],
    )
    del perm
    return pl.pallas_call(
        _spec_kernel,
        grid_spec=grid_spec,
        out_shape=[jax.ShapeDtypeStruct(batch_spectrograms.shape, jnp.float32)],
    )(order, po, needed, batch_spectrograms)[0]


def _lab_sc_body(l_hbm, out_hbm, a, b, o, sa, sb, so):
    nc = 2
    wid = lax.axis_index("s") * nc + lax.axis_index("c")

    for k in range(_ROWS_PER_W):
        r = wid * _ROWS_PER_W + k
        q = jnp.int32(_PERM_NP[k])
        for w in range(32):
            q = jnp.where(wid == w, jnp.int32(_PERM_NP[w * _ROWS_PER_W + k]), q)

        ha = pltpu.async_copy(l_hbm.at[r], a, sa)
        hb = pltpu.async_copy(l_hbm.at[q], b, sb)
        ha.wait()
        hb.wait()

        @plsc.parallel_loop(0, _LPAD // 16, unroll=4)
        def _blend(i):
            sl = pl.ds(i * 16, 16)
            o[sl] = _LAM * a[sl] + (1.0 - _LAM) * b[sl]

        pltpu.async_copy(o, out_hbm.at[r], so).wait()


def _lab_mix(batch_labels):
    B, L = batch_labels.shape
    lp = jnp.pad(batch_labels, ((0, 0), (0, _LPAD - L)))
    mesh = plsc.VectorSubcoreMesh(core_axis_name="c", subcore_axis_name="s")
    out = pl.kernel(
        _lab_sc_body,
        mesh=mesh,
        out_type=jax.ShapeDtypeStruct((B, _LPAD), jnp.float32),
        scratch_types=[
            pltpu.VMEM((_LPAD,), jnp.float32),
            pltpu.VMEM((_LPAD,), jnp.float32),
            pltpu.VMEM((_LPAD,), jnp.float32),
            pltpu.SemaphoreType.DMA,
            pltpu.SemaphoreType.DMA,
            pltpu.SemaphoreType.DMA,
        ],
    )(lp)
    return out[:, :L]


def kernel(batch_spectrograms, batch_labels):
    ol = _lab_mix(batch_labels)
    ox = _spec_mix(batch_spectrograms)
    return ox, ol


# speedup vs baseline: 1.4956x; 1.2431x over previous
import numpy as np

import jax
import jax.numpy as jnp
from jax import lax
from jax.experimental import pallas as pl
from jax.experimental.pallas import tpu as pltpu
from jax.experimental.pallas import tpu_sc as plsc

_ALPHA = 0.2
_LAM = float(np.random.RandomState(0).beta(_ALPHA, _ALPHA))

_NCHUNK = 32
_OBLK = 4

with jax.default_device(jax.devices("cpu")[0]):
    _PERM_NP = np.asarray(
        jax.random.permutation(jax.random.key(42), 64)).astype(np.int32)

_ROWS_PER_W = 2
_LPAD = 528


def _spec_kernel(order_ref, po_ref, needed_ref, x_hbm, ox_ref, buf, sems,
                 waited):
    g = pl.program_id(0)
    nrows = x_hbm.shape[0]
    rpc = nrows // _NCHUNK

    @pl.when(g == 0)
    def _():
        waited[0] = 0
        for c in range(_NCHUNK):
            pltpu.make_async_copy(
                x_hbm.at[pl.ds(c * rpc, rpc)],
                buf.at[pl.ds(c * rpc, rpc)],
                sems.at[c],
            ).start()

    need = needed_ref[g]
    w0 = waited[0]
    for c in range(_NCHUNK):
        @pl.when(jnp.logical_and(c >= w0, c <= need))
        def _(c=c):
            pltpu.make_async_copy(
                x_hbm.at[pl.ds(c * rpc, rpc)],
                buf.at[pl.ds(c * rpc, rpc)],
                sems.at[c],
            ).wait()
    waited[0] = jnp.maximum(w0, need + 1)

    base = order_ref[g] * _OBLK
    for u in range(_OBLK):
        j = po_ref[g * _OBLK + u]
        ox_ref[u, 0] = _LAM * buf[base + u, 0] + (1.0 - _LAM) * buf[j, 0]


def _spec_mix(batch_spectrograms):
    B, C, H, W = batch_spectrograms.shape
    rpc = B // _NCHUNK
    ngrp = B // _OBLK

    rows_np = np.arange(B, dtype=np.int32)
    last_chunk = np.maximum(rows_np // rpc, _PERM_NP // rpc)
    grp_last = last_chunk.reshape(ngrp, _OBLK).max(axis=1)
    order_np = np.argsort(grp_last, kind="stable").astype(np.int32)
    order = jnp.asarray(order_np)
    po_np = _PERM_NP.reshape(ngrp, _OBLK)[order_np].reshape(B)
    po = jnp.asarray(po_np)
    needed = jnp.asarray(grp_last[order_np].astype(np.int32))

    grid_spec = pltpu.PrefetchScalarGridSpec(
        num_scalar_prefetch=3,
        grid=(ngrp,),
        in_specs=[pl.BlockSpec(memory_space=pl.ANY)],
        out_specs=[pl.BlockSpec((_OBLK, C, H, W),
                                lambda g, o, p, n: (o[g], 0, 0, 0))],
        scratch_shapes=[
            pltpu.VMEM((B, C, H, W), jnp.float32),
            pltpu.SemaphoreType.DMA((_NCHUNK,)),
            pltpu.SMEM((1,), jnp.int32),
        ],
    )
    return pl.pallas_call(
        _spec_kernel,
        grid_spec=grid_spec,
        out_shape=[jax.ShapeDtypeStruct(batch_spectrograms.shape, jnp.float32)],
    )(order, po, needed, batch_spectrograms)[0]


def _lab_sc_body(l_hbm, out_hbm, a, b, o, sa, sb, so):
    nc = 2
    wid = lax.axis_index("s") * nc + lax.axis_index("c")

    for k in range(_ROWS_PER_W):
        r = wid * _ROWS_PER_W + k
        q = jnp.int32(_PERM_NP[k])
        for w in range(32):
            q = jnp.where(wid == w, jnp.int32(_PERM_NP[w * _ROWS_PER_W + k]), q)

        ha = pltpu.async_copy(l_hbm.at[r], a, sa)
        hb = pltpu.async_copy(l_hbm.at[q], b, sb)
        ha.wait()
        hb.wait()

        @plsc.parallel_loop(0, _LPAD // 16, unroll=4)
        def _blend(i):
            sl = pl.ds(i * 16, 16)
            o[sl] = _LAM * a[sl] + (1.0 - _LAM) * b[sl]

        pltpu.async_copy(o, out_hbm.at[r], so).wait()


def _lab_mix(batch_labels):
    B, L = batch_labels.shape
    lp = jnp.pad(batch_labels, ((0, 0), (0, _LPAD - L)))
    mesh = plsc.VectorSubcoreMesh(core_axis_name="c", subcore_axis_name="s")
    out = pl.kernel(
        _lab_sc_body,
        mesh=mesh,
        out_type=jax.ShapeDtypeStruct((B, _LPAD), jnp.float32),
        scratch_types=[
            pltpu.VMEM((_LPAD,), jnp.float32),
            pltpu.VMEM((_LPAD,), jnp.float32),
            pltpu.VMEM((_LPAD,), jnp.float32),
            pltpu.SemaphoreType.DMA,
            pltpu.SemaphoreType.DMA,
            pltpu.SemaphoreType.DMA,
        ],
    )(lp)
    return out[:, :L]


def kernel(batch_spectrograms, batch_labels):
    ol = _lab_mix(batch_labels)
    ox = _spec_mix(batch_spectrograms)
    return ox, ol


# speedup vs baseline: 1.5355x; 1.0267x over previous
import numpy as np

import jax
import jax.numpy as jnp
from jax import lax
from jax.experimental import pallas as pl
from jax.experimental.pallas import tpu as pltpu
from jax.experimental.pallas import tpu_sc as plsc

_ALPHA = 0.2
_LAM = float(np.random.RandomState(0).beta(_ALPHA, _ALPHA))

_NCHUNK = 32
_OBLK = 8

with jax.default_device(jax.devices("cpu")[0]):
    _PERM_NP = np.asarray(
        jax.random.permutation(jax.random.key(42), 64)).astype(np.int32)

_ROWS_PER_W = 2
_LPAD = 528


def _spec_kernel(order_ref, po_ref, needed_ref, x_hbm, ox_ref, buf, sems,
                 waited):
    g = pl.program_id(0)
    nrows = x_hbm.shape[0]
    rpc = nrows // _NCHUNK

    @pl.when(g == 0)
    def _():
        waited[0] = 0
        for c in range(_NCHUNK):
            pltpu.make_async_copy(
                x_hbm.at[pl.ds(c * rpc, rpc)],
                buf.at[pl.ds(c * rpc, rpc)],
                sems.at[c],
            ).start()

    need = needed_ref[g]
    w0 = waited[0]
    for c in range(_NCHUNK):
        @pl.when(jnp.logical_and(c >= w0, c <= need))
        def _(c=c):
            pltpu.make_async_copy(
                x_hbm.at[pl.ds(c * rpc, rpc)],
                buf.at[pl.ds(c * rpc, rpc)],
                sems.at[c],
            ).wait()
    waited[0] = jnp.maximum(w0, need + 1)

    base = order_ref[g] * _OBLK
    for u in range(_OBLK):
        j = po_ref[g * _OBLK + u]
        ox_ref[u, 0] = _LAM * buf[base + u, 0] + (1.0 - _LAM) * buf[j, 0]


def _spec_mix(batch_spectrograms):
    B, C, H, W = batch_spectrograms.shape
    rpc = B // _NCHUNK
    ngrp = B // _OBLK

    rows_np = np.arange(B, dtype=np.int32)
    last_chunk = np.maximum(rows_np // rpc, _PERM_NP // rpc)
    grp_last = last_chunk.reshape(ngrp, _OBLK).max(axis=1)
    order_np = np.argsort(grp_last, kind="stable").astype(np.int32)
    order = jnp.asarray(order_np)
    po_np = _PERM_NP.reshape(ngrp, _OBLK)[order_np].reshape(B)
    po = jnp.asarray(po_np)
    needed = jnp.asarray(grp_last[order_np].astype(np.int32))

    grid_spec = pltpu.PrefetchScalarGridSpec(
        num_scalar_prefetch=3,
        grid=(ngrp,),
        in_specs=[pl.BlockSpec(memory_space=pl.ANY)],
        out_specs=[pl.BlockSpec((_OBLK, C, H, W),
                                lambda g, o, p, n: (o[g], 0, 0, 0))],
        scratch_shapes=[
            pltpu.VMEM((B, C, H, W), jnp.float32),
            pltpu.SemaphoreType.DMA((_NCHUNK,)),
            pltpu.SMEM((1,), jnp.int32),
        ],
    )
    return pl.pallas_call(
        _spec_kernel,
        grid_spec=grid_spec,
        out_shape=[jax.ShapeDtypeStruct(batch_spectrograms.shape, jnp.float32)],
    )(order, po, needed, batch_spectrograms)[0]


def _lab_sc_body(l_hbm, out_hbm, a, b, o, sa, sb, so):
    nc = 2
    wid = lax.axis_index("s") * nc + lax.axis_index("c")

    for k in range(_ROWS_PER_W):
        r = wid * _ROWS_PER_W + k
        q = jnp.int32(_PERM_NP[k])
        for w in range(32):
            q = jnp.where(wid == w, jnp.int32(_PERM_NP[w * _ROWS_PER_W + k]), q)

        ha = pltpu.async_copy(l_hbm.at[r], a, sa)
        hb = pltpu.async_copy(l_hbm.at[q], b, sb)
        ha.wait()
        hb.wait()

        @plsc.parallel_loop(0, _LPAD // 16, unroll=4)
        def _blend(i):
            sl = pl.ds(i * 16, 16)
            o[sl] = _LAM * a[sl] + (1.0 - _LAM) * b[sl]

        pltpu.async_copy(o, out_hbm.at[r], so).wait()


def _lab_mix(batch_labels):
    B, L = batch_labels.shape
    lp = jnp.pad(batch_labels, ((0, 0), (0, _LPAD - L)))
    mesh = plsc.VectorSubcoreMesh(core_axis_name="c", subcore_axis_name="s")
    out = pl.kernel(
        _lab_sc_body,
        mesh=mesh,
        out_type=jax.ShapeDtypeStruct((B, _LPAD), jnp.float32),
        scratch_types=[
            pltpu.VMEM((_LPAD,), jnp.float32),
            pltpu.VMEM((_LPAD,), jnp.float32),
            pltpu.VMEM((_LPAD,), jnp.float32),
            pltpu.SemaphoreType.DMA,
            pltpu.SemaphoreType.DMA,
            pltpu.SemaphoreType.DMA,
        ],
    )(lp)
    return out[:, :L]


def kernel(batch_spectrograms, batch_labels):
    ol = _lab_mix(batch_labels)
    ox = _spec_mix(batch_spectrograms)
    return ox, ol
